# baseline (speedup 1.0000x reference)
import functools

import jax
import jax.numpy as jnp
from jax import lax
from jax.experimental import pallas as pl
from jax.experimental.pallas import tpu as pltpu
from jax.experimental.pallas import tpu_sc as plsc

B = 4096
S = 200
D = 64
O = 20

NC = 2
NS = 16
NW = NC * NS
ROWS_PER_W = B // NW
ROWS_PER_SC = B // NC
CHUNK_A = 128
CHUNK_B = S - CHUNK_A
SP = 256


def _sc_pool(x, dest_a2, dest_b2, table):
    mesh = plsc.VectorSubcoreMesh(core_axis_name="c", subcore_axis_name="s")

    @functools.partial(
        pl.kernel,
        out_type=jax.ShapeDtypeStruct((B, D), jnp.float32),
        mesh=mesh,
        scratch_types=[
            pltpu.VMEM((ROWS_PER_W, SP), jnp.int32),
            pltpu.VMEM((ROWS_PER_W, CHUNK_A), jnp.int32),
            pltpu.VMEM((ROWS_PER_W, CHUNK_B), jnp.int32),
            pltpu.VMEM((CHUNK_A, D), jnp.float32),
            pltpu.VMEM((CHUNK_B, D), jnp.float32),
            pltpu.VMEM((ROWS_PER_W, D), jnp.float32),
            pltpu.VMEM_SHARED((ROWS_PER_SC, D), jnp.float32),
            pltpu.SemaphoreType.DMA,
            pltpu.SemaphoreType.DMA,
            pltpu.SemaphoreType.DMA,
            pltpu.SemaphoreType.DMA,
        ],
        compiler_params=pltpu.CompilerParams(use_tc_tiling_on_sc=False),
    )
    def body(x_hbm, dest_a_hbm, dest_b_hbm, table_hbm, out_hbm,
             idx_v, dest_a, dest_b, buf0, buf1, zbuf, acc_sh, g0, g1, s0, s1):
        c = lax.axis_index("c")
        s = lax.axis_index("s")
        wid = c * NS + s
        local_base = s * ROWS_PER_W

        pltpu.sync_copy(x_hbm.at[pl.ds(wid * ROWS_PER_W, ROWS_PER_W)], idx_v)
        pltpu.sync_copy(dest_a_hbm.at[pl.ds(local_base, ROWS_PER_W)], dest_a)
        pltpu.sync_copy(dest_b_hbm.at[pl.ds(local_base, ROWS_PER_W)], dest_b)

        zero = jnp.zeros((16,), jnp.float32)

        def zero_body(i, carry):
            r = i // (D // 16)
            j = i % (D // 16)
            zbuf[r, pl.ds(j * 16, 16)] = zero
            return carry

        lax.fori_loop(0, ROWS_PER_W * (D // 16), zero_body, 0)
        pltpu.sync_copy(zbuf, acc_sh.at[pl.ds(local_base, ROWS_PER_W)])

        def gref(row, half):
            if half == 0:
                return idx_v.at[row, pl.ds(0, CHUNK_A)]
            return idx_v.at[row, pl.ds(CHUNK_A, CHUNK_B)]

        def start_gather(row, half, buf, sem):
            return pltpu.async_copy(table_hbm.at[gref(row, half)], buf, sem)

        def wait_gather(row, half, buf, sem):
            pltpu.make_async_copy(table_hbm.at[gref(row, half)], buf, sem).wait()

        def start_scatter(row, dv, buf, sem):
            return pltpu.async_copy(buf, acc_sh.at[dv.at[row]], sem, add=True)

        def wait_scatter(row, dv, buf, sem):
            pltpu.make_async_copy(buf, acc_sh.at[dv.at[row]], sem).wait()

        start_gather(0, 0, buf0, g0)

        def row_body(r, carry):
            @pl.when(r > 0)
            def _():
                wait_scatter(r - 1, dest_b, buf1, s1)

            start_gather(r, 1, buf1, g1)
            wait_gather(r, 0, buf0, g0)
            start_scatter(r, dest_a, buf0, s0)
            wait_scatter(r, dest_a, buf0, s0)
            nxt = jnp.minimum(r + 1, ROWS_PER_W - 1)
            start_gather(nxt, 0, buf0, g0)
            wait_gather(r, 1, buf1, g1)
            start_scatter(r, dest_b, buf1, s1)
            return carry

        lax.fori_loop(0, ROWS_PER_W, row_body, 0)
        wait_scatter(ROWS_PER_W - 1, dest_b, buf1, s1)
        wait_gather(ROWS_PER_W - 1, 0, buf0, g0)

        pltpu.sync_copy(
            acc_sh.at[pl.ds(local_base, ROWS_PER_W)],
            out_hbm.at[pl.ds(wid * ROWS_PER_W, ROWS_PER_W)],
        )

    return body(x, dest_a2, dest_b2, table)


def _head_body(s_ref, w_ref, b_ref, o_ref):
    rep = jnp.maximum(s_ref[...] * (1.0 / S), 0.0)
    o_ref[...] = (
        jnp.dot(rep, w_ref[...], preferred_element_type=jnp.float32)
        + b_ref[...]
    )


def _tc_head(sums, W, b):
    blk = 1024
    return pl.pallas_call(
        _head_body,
        out_shape=jax.ShapeDtypeStruct((B, O), jnp.float32),
        grid=(B // blk,),
        in_specs=[
            pl.BlockSpec((blk, D), lambda i: (i, 0)),
            pl.BlockSpec((D, O), lambda i: (0, 0)),
            pl.BlockSpec((1, O), lambda i: (0, 0)),
        ],
        out_specs=pl.BlockSpec((blk, O), lambda i: (i, 0)),
    )(sums, W, b.reshape(1, O))


def kernel(x, lengths, table, W, b):
    del lengths
    loc = jnp.arange(ROWS_PER_SC, dtype=jnp.int32)[:, None]
    dest_a2 = jnp.broadcast_to(loc, (ROWS_PER_SC, CHUNK_A))
    dest_b2 = jnp.broadcast_to(loc, (ROWS_PER_SC, CHUNK_B))
    xp = jnp.pad(x.astype(jnp.int32), ((0, 0), (0, SP - S)))
    sums = _sc_pool(xp, dest_a2, dest_b2, table)
    return _tc_head(sums, W, b)


# speedup vs baseline: 1.0072x; 1.0072x over previous
import functools

import jax
import jax.numpy as jnp
from jax import lax
from jax.experimental import pallas as pl
from jax.experimental.pallas import tpu as pltpu
from jax.experimental.pallas import tpu_sc as plsc

B = 4096
S = 200
D = 64
O = 20

NC = 2
NS = 16
NW = NC * NS
ROWS_PER_W = B // NW
ROWS_PER_SC = B // NC
CHUNK_A = 128
CHUNK_B = S - CHUNK_A
SP = 256


def _sc_pool(x, table):
    mesh = plsc.VectorSubcoreMesh(core_axis_name="c", subcore_axis_name="s")

    @functools.partial(
        pl.kernel,
        out_type=jax.ShapeDtypeStruct((B, D), jnp.float32),
        mesh=mesh,
        scratch_types=[
            pltpu.VMEM((ROWS_PER_W, SP), jnp.int32),
            pltpu.VMEM((ROWS_PER_W, CHUNK_A), jnp.int32),
            pltpu.VMEM((ROWS_PER_W, CHUNK_B), jnp.int32),
            pltpu.VMEM((CHUNK_A, D), jnp.float32),
            pltpu.VMEM((CHUNK_B, D), jnp.float32),
            pltpu.VMEM((ROWS_PER_W, D), jnp.float32),
            pltpu.VMEM_SHARED((ROWS_PER_SC, D), jnp.float32),
            pltpu.SemaphoreType.DMA,
            pltpu.SemaphoreType.DMA,
            pltpu.SemaphoreType.DMA,
            pltpu.SemaphoreType.DMA,
        ],
        compiler_params=pltpu.CompilerParams(use_tc_tiling_on_sc=False),
    )
    def body(x_hbm, table_hbm, out_hbm,
             idx_v, dest_a, dest_b, buf0, buf1, zbuf, acc_sh, g0, g1, s0, s1):
        c = lax.axis_index("c")
        s = lax.axis_index("s")
        wid = c * NS + s
        local_base = s * ROWS_PER_W

        pltpu.sync_copy(x_hbm.at[pl.ds(wid * ROWS_PER_W, ROWS_PER_W)], idx_v)

        def fill_dest(r, carry):
            v = jnp.full((16,), local_base + r, jnp.int32)
            for j in range(CHUNK_A // 16):
                dest_a[r, pl.ds(j * 16, 16)] = v
            for off in (0, 16, 32, 48, CHUNK_B - 16):
                dest_b[r, pl.ds(off, 16)] = v
            return carry

        lax.fori_loop(0, ROWS_PER_W, fill_dest, 0)

        zero = jnp.zeros((16,), jnp.float32)

        def zero_body(i, carry):
            r = i // (D // 16)
            j = i % (D // 16)
            zbuf[r, pl.ds(j * 16, 16)] = zero
            return carry

        lax.fori_loop(0, ROWS_PER_W * (D // 16), zero_body, 0)
        pltpu.sync_copy(zbuf, acc_sh.at[pl.ds(local_base, ROWS_PER_W)])

        def gref(row, half):
            if half == 0:
                return idx_v.at[row, pl.ds(0, CHUNK_A)]
            return idx_v.at[row, pl.ds(CHUNK_A, CHUNK_B)]

        def start_gather(row, half, buf, sem):
            return pltpu.async_copy(table_hbm.at[gref(row, half)], buf, sem)

        def wait_gather(row, half, buf, sem):
            pltpu.make_async_copy(table_hbm.at[gref(row, half)], buf, sem).wait()

        def start_scatter(row, dv, buf, sem):
            return pltpu.async_copy(buf, acc_sh.at[dv.at[row]], sem, add=True)

        def wait_scatter(row, dv, buf, sem):
            pltpu.make_async_copy(buf, acc_sh.at[dv.at[row]], sem).wait()

        start_gather(0, 0, buf0, g0)

        def row_body(r, carry):
            @pl.when(r > 0)
            def _():
                wait_scatter(r - 1, dest_b, buf1, s1)

            start_gather(r, 1, buf1, g1)
            wait_gather(r, 0, buf0, g0)
            start_scatter(r, dest_a, buf0, s0)
            wait_scatter(r, dest_a, buf0, s0)
            nxt = jnp.minimum(r + 1, ROWS_PER_W - 1)
            start_gather(nxt, 0, buf0, g0)
            wait_gather(r, 1, buf1, g1)
            start_scatter(r, dest_b, buf1, s1)
            return carry

        lax.fori_loop(0, ROWS_PER_W, row_body, 0)
        wait_scatter(ROWS_PER_W - 1, dest_b, buf1, s1)
        wait_gather(ROWS_PER_W - 1, 0, buf0, g0)

        pltpu.sync_copy(
            acc_sh.at[pl.ds(local_base, ROWS_PER_W)],
            out_hbm.at[pl.ds(wid * ROWS_PER_W, ROWS_PER_W)],
        )

    return body(x, table)


def _head_body(s_ref, w_ref, b_ref, o_ref):
    rep = jnp.maximum(s_ref[...] * (1.0 / S), 0.0)
    o_ref[...] = (
        jnp.dot(rep, w_ref[...], preferred_element_type=jnp.float32)
        + b_ref[...]
    )


def _tc_head(sums, W, b):
    blk = 1024
    return pl.pallas_call(
        _head_body,
        out_shape=jax.ShapeDtypeStruct((B, O), jnp.float32),
        grid=(B // blk,),
        in_specs=[
            pl.BlockSpec((blk, D), lambda i: (i, 0)),
            pl.BlockSpec((D, O), lambda i: (0, 0)),
            pl.BlockSpec((1, O), lambda i: (0, 0)),
        ],
        out_specs=pl.BlockSpec((blk, O), lambda i: (i, 0)),
    )(sums, W, b.reshape(1, O))


def kernel(x, lengths, table, W, b):
    del lengths
    xp = jnp.pad(x.astype(jnp.int32), ((0, 0), (0, SP - S)))
    sums = _sc_pool(xp, table)
    return _tc_head(sums, W, b)


# speedup vs baseline: 1.0447x; 1.0373x over previous
import functools

import jax
import jax.numpy as jnp
from jax import lax
from jax.experimental import pallas as pl
from jax.experimental.pallas import tpu as pltpu
from jax.experimental.pallas import tpu_sc as plsc

B = 4096
S = 200
D = 64
O = 20

NC = 2
NS = 16
NW = NC * NS
ROWS_PER_W = B // NW
ROWS_PER_SC = B // NC
CHUNK_A = 128
CHUNK_B = S - CHUNK_A
SP = 256
DP = 128


def _sc_pool(x, table):
    mesh = plsc.VectorSubcoreMesh(core_axis_name="c", subcore_axis_name="s")

    @functools.partial(
        pl.kernel,
        out_type=jax.ShapeDtypeStruct((B, D), jnp.float32),
        mesh=mesh,
        scratch_types=[
            pltpu.VMEM((ROWS_PER_W, SP), jnp.int32),
            pltpu.VMEM((ROWS_PER_W, CHUNK_A), jnp.int32),
            pltpu.VMEM((ROWS_PER_W, CHUNK_B), jnp.int32),
            pltpu.VMEM((CHUNK_A, D), jnp.float32),
            pltpu.VMEM((CHUNK_B, D), jnp.float32),
            pltpu.VMEM((CHUNK_A, D), jnp.float32),
            pltpu.VMEM((CHUNK_B, D), jnp.float32),
            pltpu.VMEM((ROWS_PER_W, D), jnp.float32),
            pltpu.VMEM_SHARED((ROWS_PER_SC, D), jnp.float32),
            pltpu.SemaphoreType.DMA,
            pltpu.SemaphoreType.DMA,
            pltpu.SemaphoreType.DMA,
            pltpu.SemaphoreType.DMA,
            pltpu.SemaphoreType.DMA,
            pltpu.SemaphoreType.DMA,
            pltpu.SemaphoreType.DMA,
            pltpu.SemaphoreType.DMA,
        ],
        compiler_params=pltpu.CompilerParams(use_tc_tiling_on_sc=False),
    )
    def body(x_hbm, table_hbm, out_hbm,
             idx_v, dest_a, dest_b, bufa0, bufb0, bufa1, bufb1, zbuf, acc_sh,
             g0, g1, g2, g3, sc0, sc1, sc2, sc3):
        c = lax.axis_index("c")
        s = lax.axis_index("s")
        wid = c * NS + s
        local_base = s * ROWS_PER_W

        pltpu.sync_copy(x_hbm.at[pl.ds(wid * ROWS_PER_W, ROWS_PER_W)], idx_v)

        def fill_dest(r, carry):
            v = jnp.full((16,), local_base + r, jnp.int32)
            for j in range(CHUNK_A // 16):
                dest_a[r, pl.ds(j * 16, 16)] = v
            for off in (0, 16, 32, 48, CHUNK_B - 16):
                dest_b[r, pl.ds(off, 16)] = v
            return carry

        lax.fori_loop(0, ROWS_PER_W, fill_dest, 0)

        zero = jnp.zeros((16,), jnp.float32)

        def zero_body(i, carry):
            r = i // (D // 16)
            j = i % (D // 16)
            zbuf[r, pl.ds(j * 16, 16)] = zero
            return carry

        lax.fori_loop(0, ROWS_PER_W * (D // 16), zero_body, 0)
        pltpu.sync_copy(zbuf, acc_sh.at[pl.ds(local_base, ROWS_PER_W)])

        def gref(row, half):
            if half == 0:
                return idx_v.at[row, pl.ds(0, CHUNK_A)]
            return idx_v.at[row, pl.ds(CHUNK_A, CHUNK_B)]

        def start_gather(row, half, buf, sem):
            return pltpu.async_copy(table_hbm.at[gref(row, half)], buf, sem)

        def wait_gather(row, half, buf, sem):
            pltpu.make_async_copy(table_hbm.at[gref(row, half)], buf, sem).wait()

        def start_scatter(row, dv, buf, sem):
            return pltpu.async_copy(buf, acc_sh.at[dv.at[row]], sem, add=True)

        def wait_scatter(row, dv, buf, sem):
            pltpu.make_async_copy(buf, acc_sh.at[dv.at[row]], sem).wait()

        bufs = (bufa0, bufb0, bufa1, bufb1)
        gsems = (g0, g1, g2, g3)
        ssems = (sc0, sc1, sc2, sc3)
        dests = (dest_a, dest_b)
        n_chunks = 2 * ROWS_PER_W

        start_gather(0, 0, bufs[0], gsems[0])
        start_gather(0, 1, bufs[1], gsems[1])

        def quad_body(i, carry):
            for k in range(4):
                c = 4 * i + k
                row = 2 * i + k // 2
                half = k % 2
                wait_gather(row, half, bufs[k], gsems[k])
                start_scatter(row, dests[half], bufs[k], ssems[k])
                kn = (k + 2) % 4
                nrow = row + 1

                @pl.when(c >= 2)
                def _():
                    wait_scatter(row - 1, dests[half], bufs[kn], ssems[kn])

                gr = jnp.minimum(nrow, ROWS_PER_W - 1)
                start_gather(gr, half, bufs[kn], gsems[kn])
            return carry

        lax.fori_loop(0, n_chunks // 4, quad_body, 0)
        last = ROWS_PER_W - 1
        wait_scatter(last, dest_a, bufs[2], ssems[2])
        wait_scatter(last, dest_b, bufs[3], ssems[3])
        wait_gather(last, 0, bufs[0], gsems[0])
        wait_gather(last, 1, bufs[1], gsems[1])

        pltpu.sync_copy(
            acc_sh.at[pl.ds(local_base, ROWS_PER_W)],
            out_hbm.at[pl.ds(wid * ROWS_PER_W, ROWS_PER_W)],
        )

    return body(x, table)


def _head_body(s_ref, w_ref, b_ref, o_ref):
    rep = jnp.maximum(s_ref[...] * (1.0 / S), 0.0)
    o_ref[...] = (
        jnp.dot(rep, w_ref[...], preferred_element_type=jnp.float32)
        + b_ref[...]
    )


def _tc_head(sums, W, b):
    blk = 1024
    return pl.pallas_call(
        _head_body,
        out_shape=jax.ShapeDtypeStruct((B, O), jnp.float32),
        grid=(B // blk,),
        in_specs=[
            pl.BlockSpec((blk, D), lambda i: (i, 0)),
            pl.BlockSpec((D, O), lambda i: (0, 0)),
            pl.BlockSpec((1, O), lambda i: (0, 0)),
        ],
        out_specs=pl.BlockSpec((blk, O), lambda i: (i, 0)),
    )(sums, W, b.reshape(1, O))


def kernel(x, lengths, table, W, b):
    del lengths
    xp = jnp.pad(x.astype(jnp.int32), ((0, 0), (0, SP - S)))
    sums = _sc_pool(xp, table)
    return _tc_head(sums, W, b)


# speedup vs baseline: 1.0838x; 1.0374x over previous
import functools

import jax
import jax.numpy as jnp
from jax import lax
from jax.experimental import pallas as pl
from jax.experimental.pallas import tpu as pltpu
from jax.experimental.pallas import tpu_sc as plsc

B = 4096
S = 200
D = 64
O = 20

NC = 2
NS = 16
NW = NC * NS
ROWS_PER_W = B // NW
ROWS_PER_SC = B // NC
CHUNK_A = 128
CHUNK_B = S - CHUNK_A
SP = 256
DP = 128


def _sc_pool(x, table):
    mesh = plsc.VectorSubcoreMesh(core_axis_name="c", subcore_axis_name="s")

    @functools.partial(
        pl.kernel,
        out_type=jax.ShapeDtypeStruct((B, D), jnp.float32),
        mesh=mesh,
        scratch_types=[
            pltpu.VMEM((ROWS_PER_W, SP), jnp.int32),
            pltpu.VMEM((ROWS_PER_W, CHUNK_A), jnp.int32),
            pltpu.VMEM((ROWS_PER_W, CHUNK_B), jnp.int32),
            pltpu.VMEM((CHUNK_A, D), jnp.float32),
            pltpu.VMEM((CHUNK_B, D), jnp.float32),
            pltpu.VMEM((CHUNK_A, D), jnp.float32),
            pltpu.VMEM((CHUNK_B, D), jnp.float32),
            pltpu.VMEM((CHUNK_A, D), jnp.float32),
            pltpu.VMEM((CHUNK_B, D), jnp.float32),
            pltpu.VMEM((CHUNK_A, D), jnp.float32),
            pltpu.VMEM((CHUNK_B, D), jnp.float32),
            pltpu.VMEM((ROWS_PER_W, D), jnp.float32),
            pltpu.VMEM_SHARED((ROWS_PER_SC, D), jnp.float32),
        ] + [pltpu.SemaphoreType.DMA] * 16,
        compiler_params=pltpu.CompilerParams(use_tc_tiling_on_sc=False),
    )
    def body(x_hbm, table_hbm, out_hbm,
             idx_v, dest_a, dest_b, bufa0, bufb0, bufa1, bufb1,
             bufa2, bufb2, bufa3, bufb3, zbuf, acc_sh, *sems):
        c = lax.axis_index("c")
        s = lax.axis_index("s")
        wid = c * NS + s
        local_base = s * ROWS_PER_W

        pltpu.sync_copy(x_hbm.at[pl.ds(wid * ROWS_PER_W, ROWS_PER_W)], idx_v)

        def fill_dest(r, carry):
            v = jnp.full((16,), local_base + r, jnp.int32)
            for j in range(CHUNK_A // 16):
                dest_a[r, pl.ds(j * 16, 16)] = v
            for off in (0, 16, 32, 48, CHUNK_B - 16):
                dest_b[r, pl.ds(off, 16)] = v
            return carry

        lax.fori_loop(0, ROWS_PER_W, fill_dest, 0)

        zero = jnp.zeros((16,), jnp.float32)

        def zero_body(i, carry):
            r = i // (D // 16)
            j = i % (D // 16)
            zbuf[r, pl.ds(j * 16, 16)] = zero
            return carry

        lax.fori_loop(0, ROWS_PER_W * (D // 16), zero_body, 0)
        pltpu.sync_copy(zbuf, acc_sh.at[pl.ds(local_base, ROWS_PER_W)])

        def gref(row, half):
            if half == 0:
                return idx_v.at[row, pl.ds(0, CHUNK_A)]
            return idx_v.at[row, pl.ds(CHUNK_A, CHUNK_B)]

        def start_gather(row, half, buf, sem):
            return pltpu.async_copy(table_hbm.at[gref(row, half)], buf, sem)

        def wait_gather(row, half, buf, sem):
            pltpu.make_async_copy(table_hbm.at[gref(row, half)], buf, sem).wait()

        def start_scatter(row, dv, buf, sem):
            return pltpu.async_copy(buf, acc_sh.at[dv.at[row]], sem, add=True)

        def wait_scatter(row, dv, buf, sem):
            pltpu.make_async_copy(buf, acc_sh.at[dv.at[row]], sem).wait()

        bufs = (bufa0, bufb0, bufa1, bufb1, bufa2, bufb2, bufa3, bufb3)
        gsems = sems[:8]
        ssems = sems[8:]
        dests = (dest_a, dest_b)
        n_chunks = 2 * ROWS_PER_W
        last = ROWS_PER_W - 1

        for k in range(4):
            start_gather(k // 2, k % 2, bufs[k], gsems[k])

        def oct_body(i, carry):
            for k in range(8):
                c = 8 * i + k
                row = 4 * i + k // 2
                half = k % 2
                wait_gather(row, half, bufs[k], gsems[k])
                start_scatter(row, dests[half], bufs[k], ssems[k])
                kn = (k + 4) % 8

                @pl.when(c >= 4)
                def _():
                    wait_scatter(row - 2, dests[half], bufs[kn], ssems[kn])

                gr = jnp.minimum(row + 2, last)
                start_gather(gr, half, bufs[kn], gsems[kn])
            return carry

        lax.fori_loop(0, n_chunks // 8, oct_body, 0)
        for k in range(4, 8):
            wait_scatter(last - 1 + (k - 4) // 2, dests[k % 2],
                         bufs[k], ssems[k])
        for k in range(4):
            wait_gather(last, k % 2, bufs[k], gsems[k])

        pltpu.sync_copy(
            acc_sh.at[pl.ds(local_base, ROWS_PER_W)],
            out_hbm.at[pl.ds(wid * ROWS_PER_W, ROWS_PER_W)],
        )

    return body(x, table)


def _head_body(s_ref, w_ref, b_ref, o_ref):
    rep = jnp.maximum(s_ref[...] * (1.0 / S), 0.0)
    o_ref[...] = (
        jnp.dot(rep, w_ref[...], preferred_element_type=jnp.float32)
        + b_ref[...]
    )


def _tc_head(sums, W, b):
    blk = 1024
    return pl.pallas_call(
        _head_body,
        out_shape=jax.ShapeDtypeStruct((B, O), jnp.float32),
        grid=(B // blk,),
        in_specs=[
            pl.BlockSpec((blk, D), lambda i: (i, 0)),
            pl.BlockSpec((D, O), lambda i: (0, 0)),
            pl.BlockSpec((1, O), lambda i: (0, 0)),
        ],
        out_specs=pl.BlockSpec((blk, O), lambda i: (i, 0)),
    )(sums, W, b.reshape(1, O))


def kernel(x, lengths, table, W, b):
    del lengths
    xp = jnp.pad(x.astype(jnp.int32), ((0, 0), (0, SP - S)))
    sums = _sc_pool(xp, table)
    return _tc_head(sums, W, b)
